# baseline (speedup 1.0000x reference)
import functools

import jax
import jax.numpy as jnp
import numpy as np
from jax import lax
from jax.experimental import pallas as pl
from jax.experimental.pallas import tpu as pltpu
from jax.experimental.pallas import tpu_sc as plsc

_H, _W = 240, 320
_C = 9
_B = 4
_N = 2000000
_HW = _H * _W

_TCCHUNK = 80000
_CHUNK = 2000
_NCHUNKS = _N // _CHUNK
_GROUPS = _CHUNK // 16
_SLAB = 3840
_ROLES_PER_B = _HW // _SLAB
_NCH = 30
_ACCW = _NCH * _SLAB

_VG_OFF = 18 * _SLAB
_EF_OFF = 27 * _SLAB
_EC0_OFF = 28 * _SLAB
_EC1_OFF = 29 * _SLAB


def _boundspass_body(ev_ref, bnd_ref):
    i = pl.program_id(0)
    t = ev_ref[2, :]
    b = ev_ref[4, :]

    @pl.when(i == 0)
    def _init():
        for j in range(16):
            bnd_ref[j] = 0.0

    bnd_ref[0] = jnp.maximum(bnd_ref[0], jnp.max(t))
    bnd_ref[1] = bnd_ref[1] + jnp.sum((b < 1.0).astype(jnp.float32))
    bnd_ref[2] = bnd_ref[2] + jnp.sum((b < 2.0).astype(jnp.float32))
    bnd_ref[3] = bnd_ref[3] + jnp.sum((b < 3.0).astype(jnp.float32))


def _packpass_body(ev_ref, bnd_ref, key_ref, v0_ref, v1_ref):
    x = ev_ref[0, :]
    y = ev_ref[1, :]
    t = ev_ref[2, :]
    p = ev_ref[3, :]
    b = ev_ref[4, :]
    tmax = bnd_ref[0]
    tn = t / tmax
    jf = jnp.floor(tn * 8.0)
    ts0 = tn - jf * 0.125
    ts1 = tn - (jf + 1.0) * 0.125
    w0 = jnp.where(ts0 > 0.0, 1.0 - 8.0 * ts0, 0.0)
    w1 = jnp.where(ts1 < 0.0, 8.0 * ts1 + 1.0, 0.0)
    v0_ref[0, 0, :] = tn * w0
    v1_ref[0, 0, :] = tn * w1
    cf = jnp.clip(jnp.floor(tn * 9.0), 0.0, 8.0)
    g_lo = cf / 9.0
    g_hi = (cf + 1.0) / 9.0
    cf = jnp.where(tn <= g_lo, cf - 1.0, jnp.where(tn > g_hi, cf + 1.0, cf))
    s = (x + 320.0 * y).astype(jnp.int32)
    key = (s + b.astype(jnp.int32) * 131072 + p.astype(jnp.int32) * 524288
           + jf.astype(jnp.int32) * 1048576 + cf.astype(jnp.int32) * 16777216)
    key_ref[0, 0, :] = key


def _prepass(events):
    ev_t = events.T
    bounds = pl.pallas_call(
        _boundspass_body,
        grid=(_N // _TCCHUNK,),
        in_specs=[pl.BlockSpec((5, _TCCHUNK), lambda i: (0, i))],
        out_specs=pl.BlockSpec(memory_space=pltpu.MemorySpace.SMEM),
        out_shape=jax.ShapeDtypeStruct((16,), jnp.float32),
    )(ev_t)
    keys, v0, v1 = pl.pallas_call(
        _packpass_body,
        grid=(_N // _TCCHUNK,),
        in_specs=[
            pl.BlockSpec((5, _TCCHUNK), lambda i: (0, i)),
            pl.BlockSpec(memory_space=pltpu.MemorySpace.SMEM),
        ],
        out_specs=[
            pl.BlockSpec((1, 1, _TCCHUNK), lambda i: (i, 0, 0)),
            pl.BlockSpec((1, 1, _TCCHUNK), lambda i: (i, 0, 0)),
            pl.BlockSpec((1, 1, _TCCHUNK), lambda i: (i, 0, 0)),
        ],
        out_shape=[
            jax.ShapeDtypeStruct((_N // _TCCHUNK, 1, _TCCHUNK), jnp.int32),
            jax.ShapeDtypeStruct((_N // _TCCHUNK, 1, _TCCHUNK), jnp.float32),
            jax.ShapeDtypeStruct((_N // _TCCHUNK, 1, _TCCHUNK), jnp.float32),
        ],
    )(ev_t, bounds)
    return keys, v0, v1, bounds


def _sc_body(keys_hbm, v0_hbm, v1_hbm, bnd_hbm, out_hbm,
             kbuf, abuf, bbuf, acc, bndbuf, sem):
    cid = lax.axis_index("c")
    sid = lax.axis_index("s")
    wid = sid * 2 + cid

    lidx = lax.iota(jnp.int32, 16)
    zeros = jnp.zeros((16,), jnp.float32)
    ones = jnp.ones((16,), jnp.float32)

    pltpu.sync_copy(bnd_hbm, bndbuf)
    bndv = bndbuf[...]
    b1 = jnp.max(jnp.where(lidx == 1, bndv, 0.0))
    b2 = jnp.max(jnp.where(lidx == 2, bndv, 0.0))
    b3 = jnp.max(jnp.where(lidx == 3, bndv, 0.0))

    n_roles = _B * _ROLES_PER_B

    def pass_body(pnum, _):
        role = pnum * 32 + wid

        @pl.when(role < n_roles)
        def _active():
            _do_role(role)
        return 0

    def _do_role(role):
        batch = role // _ROLES_PER_B
        slab_start = (role - batch * _ROLES_PER_B) * _SLAB
        memb_base = batch * 131072 + slab_start

        def zero_body(j, _):
            acc[pl.ds(j * 16, 16)] = zeros
            return 0

        lax.fori_loop(0, _ACCW // 16, zero_body, 0)

        start_f = jnp.where(
            batch == 0, 0.0,
            jnp.where(batch == 1, b1, jnp.where(batch == 2, b2, b3)))
        end_f = jnp.where(
            batch == 0, b1,
            jnp.where(batch == 1, b2, jnp.where(batch == 2, b3, float(_N))))
        c_lo = jnp.maximum(
            (start_f * (1.0 / _CHUNK)).astype(jnp.int32) - 1, 0)
        c_hi = jnp.minimum(
            (end_f * (1.0 / _CHUNK)).astype(jnp.int32) + 2, _NCHUNKS)

        def start_fetch(ci, slot):
            src = pl.ds(ci * _CHUNK, _CHUNK)
            dst = pl.ds(slot * _CHUNK, _CHUNK)
            pltpu.async_copy(keys_hbm.at[src], kbuf.at[dst], sem)
            pltpu.async_copy(v0_hbm.at[src], abuf.at[dst], sem)
            pltpu.async_copy(v1_hbm.at[src], bbuf.at[dst], sem)

        start_fetch(c_lo, 0)

        def chunk_body(ci_rel, _):
            ci = c_lo + ci_rel
            slot = ci_rel & 1
            boff = slot * _CHUNK

            @pl.when(ci + 1 < c_hi)
            def _prefetch():
                start_fetch(ci + 1, 1 - slot)

            dst = pl.ds(boff, _CHUNK)
            src0 = pl.ds(0, _CHUNK)
            pltpu.make_async_copy(keys_hbm.at[src0], kbuf.at[dst], sem).wait()
            pltpu.make_async_copy(v0_hbm.at[src0], abuf.at[dst], sem).wait()
            pltpu.make_async_copy(v1_hbm.at[src0], bbuf.at[dst], sem).wait()

            def group_body(g, _):
                off = pl.ds(boff + g * 16, 16)
                key = kbuf[off]
                va = abuf[off]
                vb = bbuf[off]
                diff = (key & 0x7FFFF) - memb_base
                m = diff.astype(jnp.uint32) < _SLAB
                pi = (key >> 19) & 1
                jfi = (key >> 20) & 15
                cvg = key >> 24
                p32 = pi * _SLAB
                idx0 = p32 * 9 + jfi * _SLAB + diff
                idxvg = cvg * _SLAB + (diff + _VG_OFF)
                idxec = p32 + (diff + _EC0_OFF)
                plsc.addupdate_scatter(acc, [idx0], va, mask=m)
                plsc.addupdate_scatter(acc, [idx0 + _SLAB], vb, mask=m)
                plsc.addupdate_scatter(acc, [idxvg], ones, mask=m)
                plsc.addupdate_scatter(acc, [idxec], ones, mask=m)
                return 0

            lax.fori_loop(0, _GROUPS, group_body, 0, unroll=25)
            return 0

        lax.fori_loop(0, c_hi - c_lo, chunk_body, 0)

        def vgfin(j, _):
            off = _VG_OFF + j * 16
            v = acc[pl.ds(off, 16)]
            acc[pl.ds(off, 16)] = jnp.where(v > 0.0, 1.0, v)
            return 0

        lax.fori_loop(0, 9 * _SLAB // 16, vgfin, 0)

        def effin(j, _):
            o = j * 16
            acc[pl.ds(_EF_OFF + o, 16)] = (
                acc[pl.ds(_EC0_OFF + o, 16)] + acc[pl.ds(_EC1_OFF + o, 16)])
            return 0

        lax.fori_loop(0, _SLAB // 16, effin, 0)

        row0 = batch * _NCH
        for ch in range(_NCH):
            pltpu.sync_copy(
                acc.at[pl.ds(ch * _SLAB, _SLAB)],
                out_hbm.at[row0 + ch, pl.ds(slab_start, _SLAB)])
        return 0

    lax.fori_loop(0, 3, pass_body, 0)


def _make_sc_kernel():
    mesh = plsc.VectorSubcoreMesh(core_axis_name="c", subcore_axis_name="s")
    return functools.partial(
        pl.kernel,
        mesh=mesh,
        compiler_params=pltpu.CompilerParams(needs_layout_passes=False),
        out_type=jax.ShapeDtypeStruct((_B * _NCH, _HW), jnp.float32),
        scratch_types=[
            pltpu.VMEM((2 * _CHUNK,), jnp.int32),
            pltpu.VMEM((2 * _CHUNK,), jnp.float32),
            pltpu.VMEM((2 * _CHUNK,), jnp.float32),
            pltpu.VMEM((_ACCW,), jnp.float32),
            pltpu.VMEM((16,), jnp.float32),
            pltpu.SemaphoreType.DMA,
        ],
    )


def kernel(events):
    keys, v0, v1, bounds = _prepass(events)
    sc = _make_sc_kernel()(_sc_body)
    out = sc(keys.reshape(_N), v0.reshape(_N), v1.reshape(_N), bounds)
    return out.reshape(_B, _NCH, _H, _W)


# speedup vs baseline: 1.0118x; 1.0118x over previous
import functools

import jax
import jax.numpy as jnp
import numpy as np
from jax import lax
from jax.experimental import pallas as pl
from jax.experimental.pallas import tpu as pltpu
from jax.experimental.pallas import tpu_sc as plsc

_H, _W = 240, 320
_C = 9
_B = 4
_N = 2000000
_HW = _H * _W

_TCCHUNK = 80000
_CHUNK = 2000
_NCHUNKS = _N // _CHUNK
_GROUPS = _CHUNK // 16
_SLAB = 3200
_ROLES_PER_B = _HW // _SLAB
_NCH = 30
_ACCW = _NCH * _SLAB

_VG_OFF = 18 * _SLAB
_EF_OFF = 27 * _SLAB
_EC0_OFF = 28 * _SLAB
_EC1_OFF = 29 * _SLAB


def _boundspass_body(ev_ref, bnd_ref):
    i = pl.program_id(0)
    t = ev_ref[2, :]
    b = ev_ref[4, :]

    @pl.when(i == 0)
    def _init():
        for j in range(16):
            bnd_ref[j] = 0.0

    bnd_ref[0] = jnp.maximum(bnd_ref[0], jnp.max(t))
    bnd_ref[1] = bnd_ref[1] + jnp.sum((b < 1.0).astype(jnp.float32))
    bnd_ref[2] = bnd_ref[2] + jnp.sum((b < 2.0).astype(jnp.float32))
    bnd_ref[3] = bnd_ref[3] + jnp.sum((b < 3.0).astype(jnp.float32))


def _packpass_body(ev_ref, bnd_ref, key_ref, v0_ref, v1_ref):
    x = ev_ref[0, :]
    y = ev_ref[1, :]
    t = ev_ref[2, :]
    p = ev_ref[3, :]
    b = ev_ref[4, :]
    tmax = bnd_ref[0]
    tn = t / tmax
    jf = jnp.floor(tn * 8.0)
    ts0 = tn - jf * 0.125
    ts1 = tn - (jf + 1.0) * 0.125
    w0 = jnp.where(ts0 > 0.0, 1.0 - 8.0 * ts0, 0.0)
    w1 = jnp.where(ts1 < 0.0, 8.0 * ts1 + 1.0, 0.0)
    v0_ref[0, 0, :] = tn * w0
    v1_ref[0, 0, :] = tn * w1
    cf = jnp.clip(jnp.floor(tn * 9.0), 0.0, 8.0)
    g_lo = cf / 9.0
    g_hi = (cf + 1.0) / 9.0
    cf = jnp.where(tn <= g_lo, cf - 1.0, jnp.where(tn > g_hi, cf + 1.0, cf))
    s = (x + 320.0 * y).astype(jnp.int32)
    key = (s + b.astype(jnp.int32) * 131072 + p.astype(jnp.int32) * 524288
           + jf.astype(jnp.int32) * 1048576 + cf.astype(jnp.int32) * 16777216)
    key_ref[0, 0, :] = key


def _prepass(events):
    ev_t = events.T
    bounds = pl.pallas_call(
        _boundspass_body,
        grid=(_N // _TCCHUNK,),
        in_specs=[pl.BlockSpec((5, _TCCHUNK), lambda i: (0, i))],
        out_specs=pl.BlockSpec(memory_space=pltpu.MemorySpace.SMEM),
        out_shape=jax.ShapeDtypeStruct((16,), jnp.float32),
    )(ev_t)
    keys, v0, v1 = pl.pallas_call(
        _packpass_body,
        grid=(_N // _TCCHUNK,),
        in_specs=[
            pl.BlockSpec((5, _TCCHUNK), lambda i: (0, i)),
            pl.BlockSpec(memory_space=pltpu.MemorySpace.SMEM),
        ],
        out_specs=[
            pl.BlockSpec((1, 1, _TCCHUNK), lambda i: (i, 0, 0)),
            pl.BlockSpec((1, 1, _TCCHUNK), lambda i: (i, 0, 0)),
            pl.BlockSpec((1, 1, _TCCHUNK), lambda i: (i, 0, 0)),
        ],
        out_shape=[
            jax.ShapeDtypeStruct((_N // _TCCHUNK, 1, _TCCHUNK), jnp.int32),
            jax.ShapeDtypeStruct((_N // _TCCHUNK, 1, _TCCHUNK), jnp.float32),
            jax.ShapeDtypeStruct((_N // _TCCHUNK, 1, _TCCHUNK), jnp.float32),
        ],
    )(ev_t, bounds)
    return keys, v0, v1, bounds


def _sc_body(keys_hbm, v0_hbm, v1_hbm, bnd_hbm, out_hbm,
             kbuf, abuf, bbuf, acc, bndbuf, sem):
    cid = lax.axis_index("c")
    sid = lax.axis_index("s")
    wid = sid * 2 + cid

    lidx = lax.iota(jnp.int32, 16)
    zeros = jnp.zeros((16,), jnp.float32)
    ones = jnp.ones((16,), jnp.float32)

    pltpu.sync_copy(bnd_hbm, bndbuf)
    bndv = bndbuf[...]
    b1 = jnp.max(jnp.where(lidx == 1, bndv, 0.0))
    b2 = jnp.max(jnp.where(lidx == 2, bndv, 0.0))
    b3 = jnp.max(jnp.where(lidx == 3, bndv, 0.0))

    def pass_body(pnum, _):
        role = pnum * 32 + wid
        batch = role // _ROLES_PER_B
        slab_start = (role - batch * _ROLES_PER_B) * _SLAB
        memb_base = batch * 131072 + slab_start

        def zero_body(j, _):
            acc[pl.ds(j * 16, 16)] = zeros
            return 0

        lax.fori_loop(0, _ACCW // 16, zero_body, 0)

        start_f = jnp.where(
            batch == 0, 0.0,
            jnp.where(batch == 1, b1, jnp.where(batch == 2, b2, b3)))
        end_f = jnp.where(
            batch == 0, b1,
            jnp.where(batch == 1, b2, jnp.where(batch == 2, b3, float(_N))))
        c_lo = jnp.maximum(
            (start_f * (1.0 / _CHUNK)).astype(jnp.int32) - 1, 0)
        c_hi = jnp.minimum(
            (end_f * (1.0 / _CHUNK)).astype(jnp.int32) + 2, _NCHUNKS)

        def start_fetch(ci, slot):
            src = pl.ds(ci * _CHUNK, _CHUNK)
            dst = pl.ds(slot * _CHUNK, _CHUNK)
            pltpu.async_copy(keys_hbm.at[src], kbuf.at[dst], sem)
            pltpu.async_copy(v0_hbm.at[src], abuf.at[dst], sem)
            pltpu.async_copy(v1_hbm.at[src], bbuf.at[dst], sem)

        start_fetch(c_lo, 0)

        def chunk_body(ci_rel, _):
            ci = c_lo + ci_rel
            slot = ci_rel & 1
            boff = slot * _CHUNK

            @pl.when(ci + 1 < c_hi)
            def _prefetch():
                start_fetch(ci + 1, 1 - slot)

            dst = pl.ds(boff, _CHUNK)
            src0 = pl.ds(0, _CHUNK)
            pltpu.make_async_copy(keys_hbm.at[src0], kbuf.at[dst], sem).wait()
            pltpu.make_async_copy(v0_hbm.at[src0], abuf.at[dst], sem).wait()
            pltpu.make_async_copy(v1_hbm.at[src0], bbuf.at[dst], sem).wait()

            def group_body(g, _):
                off = pl.ds(boff + g * 16, 16)
                key = kbuf[off]
                va = abuf[off]
                vb = bbuf[off]
                diff = (key & 0x7FFFF) - memb_base
                m = diff.astype(jnp.uint32) < _SLAB
                pi = (key >> 19) & 1
                jfi = (key >> 20) & 15
                cvg = key >> 24
                p32 = pi * _SLAB
                idx0 = p32 * 9 + jfi * _SLAB + diff
                idxvg = cvg * _SLAB + (diff + _VG_OFF)
                idxec = p32 + (diff + _EC0_OFF)
                plsc.addupdate_scatter(acc, [idx0], va, mask=m)
                plsc.addupdate_scatter(acc, [idx0 + _SLAB], vb, mask=m)
                plsc.addupdate_scatter(acc, [idxvg], ones, mask=m)
                plsc.addupdate_scatter(acc, [idxec], ones, mask=m)
                return 0

            lax.fori_loop(0, _GROUPS, group_body, 0, unroll=25)
            return 0

        lax.fori_loop(0, c_hi - c_lo, chunk_body, 0)

        def vgfin(j, _):
            off = _VG_OFF + j * 16
            v = acc[pl.ds(off, 16)]
            acc[pl.ds(off, 16)] = jnp.where(v > 0.0, 1.0, v)
            return 0

        lax.fori_loop(0, 9 * _SLAB // 16, vgfin, 0)

        def effin(j, _):
            o = j * 16
            acc[pl.ds(_EF_OFF + o, 16)] = (
                acc[pl.ds(_EC0_OFF + o, 16)] + acc[pl.ds(_EC1_OFF + o, 16)])
            return 0

        lax.fori_loop(0, _SLAB // 16, effin, 0)

        row0 = batch * _NCH
        for ch in range(_NCH):
            pltpu.sync_copy(
                acc.at[pl.ds(ch * _SLAB, _SLAB)],
                out_hbm.at[row0 + ch, pl.ds(slab_start, _SLAB)])
        return 0

    lax.fori_loop(0, 3, pass_body, 0)


def _make_sc_kernel():
    mesh = plsc.VectorSubcoreMesh(core_axis_name="c", subcore_axis_name="s")
    return functools.partial(
        pl.kernel,
        mesh=mesh,
        compiler_params=pltpu.CompilerParams(needs_layout_passes=False),
        out_type=jax.ShapeDtypeStruct((_B * _NCH, _HW), jnp.float32),
        scratch_types=[
            pltpu.VMEM((2 * _CHUNK,), jnp.int32),
            pltpu.VMEM((2 * _CHUNK,), jnp.float32),
            pltpu.VMEM((2 * _CHUNK,), jnp.float32),
            pltpu.VMEM((_ACCW,), jnp.float32),
            pltpu.VMEM((16,), jnp.float32),
            pltpu.SemaphoreType.DMA,
        ],
    )


def kernel(events):
    keys, v0, v1, bounds = _prepass(events)
    sc = _make_sc_kernel()(_sc_body)
    out = sc(keys.reshape(_N), v0.reshape(_N), v1.reshape(_N), bounds)
    return out.reshape(_B, _NCH, _H, _W)
